# baseline (speedup 1.0000x reference)
import functools
import jax
import jax.numpy as jnp
from jax import lax
from jax.experimental import pallas as pl
from jax.experimental.pallas import tpu as pltpu
from jax.experimental.pallas import tpu_sc as plsc

_BATCH, _SEQ, _EMB = 4, 8192, 1024
_NC, _NS = 2, 16
_NW = _NC * _NS
_ROWS = _BATCH * _SEQ
_RPW = _ROWS // _NW
_CHUNK = 32
_NCH = _RPW // _CHUNK
_CW = _CHUNK * _EMB
_UNROLL = 16
_GROUPS = _CW // (16 * _UNROLL)


def _sc_body(x_hbm, w_hbm, o_hbm, xb, wb, sem_x, sem_w):
    wid = lax.axis_index("s") * _NC + lax.axis_index("c")
    base = wid * _RPW

    def chunk_body(i, carry):
        r0 = base + i * _CHUNK
        off_x = r0 * _EMB
        off_w = lax.rem(r0, _SEQ) * _EMB
        cx = pltpu.async_copy(x_hbm.at[pl.ds(off_x, _CW)], xb, sem_x)
        cw = pltpu.async_copy(w_hbm.at[pl.ds(off_w, _CW)], wb, sem_w)
        cx.wait()
        cw.wait()

        def add_body(g, c):
            b0 = g * (16 * _UNROLL)
            for u in range(_UNROLL):
                o = b0 + u * 16
                xb[pl.ds(o, 16)] = xb[pl.ds(o, 16)] + wb[pl.ds(o, 16)]
            return c

        lax.fori_loop(0, _GROUPS, add_body, 0)
        pltpu.sync_copy(xb, o_hbm.at[pl.ds(off_x, _CW)])
        return carry

    lax.fori_loop(0, _NCH, chunk_body, 0)


def kernel(x, pos_embedding_weight):
    xf = x.reshape(_ROWS * _EMB)
    wf = pos_embedding_weight.reshape(-1)[: _SEQ * _EMB]
    mesh = plsc.VectorSubcoreMesh(core_axis_name="c", subcore_axis_name="s")
    out = pl.kernel(
        _sc_body,
        mesh=mesh,
        out_type=jax.ShapeDtypeStruct((_ROWS * _EMB,), jnp.float32),
        scratch_types=[
            pltpu.VMEM((_CW,), jnp.float32),
            pltpu.VMEM((_CW,), jnp.float32),
            pltpu.SemaphoreType.DMA,
            pltpu.SemaphoreType.DMA,
        ],
    )(xf, wf)
    return out.reshape(x.shape)


# speedup vs baseline: 1.2518x; 1.2518x over previous
import jax
import jax.numpy as jnp
from jax import lax
from jax.experimental import pallas as pl
from jax.experimental.pallas import tpu as pltpu
from jax.experimental.pallas import tpu_sc as plsc

_BATCH, _SEQ, _EMB = 4, 8192, 1024
_NC, _NS = 2, 16
_NW = _NC * _NS
_SPW = _SEQ // _NW
_CH = 16
_NCH = _SPW // _CH
_CW = _CH * _EMB
_UNROLL = 16
_GROUPS = _CW // (16 * _UNROLL)


def _sc_body(x_hbm, w_hbm, o_hbm,
             xb0, xb1, xb2, xb3, wb0, wb1,
             sx0, sx1, sx2, sx3, so0, so1, so2, so3, sw0, sw1):
    xb = (xb0, xb1, xb2, xb3)
    sx = (sx0, sx1, sx2, sx3)
    so = (so0, so1, so2, so3)
    wb = (wb0, wb1)
    sw = (sw0, sw1)

    wid = lax.axis_index("s") * _NC + lax.axis_index("c")
    pos_w = wid * _SPW

    def w_off(c):
        return (pos_w + c * _CH) * _EMB

    def x_off(c, b):
        return (b * _SEQ + pos_w + c * _CH) * _EMB

    def start_load_w(c):
        return pltpu.async_copy(
            w_hbm.at[pl.ds(w_off(c), _CW)], wb[c % 2], sw[c % 2])

    def start_load_x(c, b):
        return pltpu.async_copy(
            x_hbm.at[pl.ds(x_off(c, b), _CW)], xb[b], sx[b])

    def start_store(c, b):
        return pltpu.async_copy(
            xb[b], o_hbm.at[pl.ds(x_off(c, b), _CW)], so[b])

    def add_chunk(b, wcur):
        xref = xb[b]

        def add_body(g, carry):
            base = g * (16 * _UNROLL)
            for u in range(_UNROLL):
                o = base + u * 16
                xref[pl.ds(o, 16)] = xref[pl.ds(o, 16)] + wcur[pl.ds(o, 16)]
            return carry

        lax.fori_loop(0, _GROUPS, add_body, 0)

    cw_pending = start_load_w(0)
    cx_pending = [start_load_x(0, b) for b in range(4)]
    cs_pending = [None, None, None, None]

    for c in range(_NCH):
        if c + 1 < _NCH:
            cw_next = start_load_w(c + 1)
        cw_pending.wait()
        wcur = wb[c % 2]
        for b in range(4):
            cx_pending[b].wait()
            add_chunk(b, wcur)
            cs_pending[b] = start_store(c, b)
            if c + 1 < _NCH and b >= 2:
                rb = b - 2
                cs_pending[rb].wait()
                cx_pending[rb] = start_load_x(c + 1, rb)
        if c + 1 < _NCH:
            for rb in (2, 3):
                cs_pending[rb].wait()
                cx_pending[rb] = start_load_x(c + 1, rb)
            cw_pending = cw_next
    for b in range(4):
        cs_pending[b].wait()


def kernel(x, pos_embedding_weight):
    xf = x.reshape(_BATCH * _SEQ * _EMB)
    wf = pos_embedding_weight.reshape(-1)[: _SEQ * _EMB]
    mesh = plsc.VectorSubcoreMesh(core_axis_name="c", subcore_axis_name="s")
    out = pl.kernel(
        _sc_body,
        mesh=mesh,
        out_type=jax.ShapeDtypeStruct((_BATCH * _SEQ * _EMB,), jnp.float32),
        scratch_types=(
            [pltpu.VMEM((_CW,), jnp.float32) for _ in range(4)]
            + [pltpu.VMEM((_CW,), jnp.float32) for _ in range(2)]
            + [pltpu.SemaphoreType.DMA for _ in range(10)]
        ),
    )(xf, wf)
    return out.reshape(x.shape)


# speedup vs baseline: 5.3536x; 4.2767x over previous
import jax
import jax.numpy as jnp
from jax.experimental import pallas as pl


_SEQ_BLOCK = 512


def _add_kernel(x_ref, w_ref, o_ref):
    o_ref[...] = x_ref[...] + w_ref[...][None, :, :]


def kernel(x, pos_embedding_weight):
    batch, seq_len, emb_dim = x.shape
    sb = _SEQ_BLOCK
    grid = (seq_len // sb,)
    return pl.pallas_call(
        _add_kernel,
        grid=grid,
        in_specs=[
            pl.BlockSpec((batch, sb, emb_dim), lambda s: (0, s, 0)),
            pl.BlockSpec((sb, emb_dim), lambda s: (s, 0)),
        ],
        out_specs=pl.BlockSpec((batch, sb, emb_dim), lambda s: (0, s, 0)),
        out_shape=jax.ShapeDtypeStruct(x.shape, x.dtype),
    )(x, pos_embedding_weight[:seq_len])
